# baseline (speedup 1.0000x reference)
import jax
import jax.numpy as jnp
from jax.experimental import pallas as pl

_P = 0.1
_B, _L, _V = 16384, 20, 128


def _body(f_ref, pe_ref, c_ref, m_ref, e_ref,
          o_ref, sym_ref, me_ref, mn_ref):
    l = pl.program_id(0)
    m = m_ref[0]
    f = f_ref[0, 0]
    pe = pe_ref[0, 0]
    c = c_ref[0, 0]
    lane = jax.lax.broadcasted_iota(jnp.int32, (1, _V), 1)
    scale = jnp.where(lane == 0, 1.0, f)
    t = jnp.transpose(m * scale)
    o_ref[0, : _V, :] = t
    o_ref[0, _V:, :] = pe * (1.0 - t[:1, :])

    e = e_ref[0]
    sym = e + c
    sym_ref[0] = sym
    me_ref[0] = jnp.where(l == 0, sym, me_ref[0] + sym)
    mn_ref[0] = jnp.where(l == 0, e, mn_ref[0] + e)


def kernel(messages, apply_noise, entropy):
    p = jnp.float32(_P)
    h2 = -p * jnp.log2(p) - (1.0 - p) * jnp.log2(1.0 - p)
    an = jnp.asarray(apply_noise)
    f = jnp.where(an, 1.0 - p, 1.0).astype(jnp.float32).reshape(1, 1)
    pe = jnp.where(an, p, 0.0).astype(jnp.float32).reshape(1, 1)
    c = jnp.where(an, h2, 0.0).astype(jnp.float32).reshape(1, 1)

    mt = jnp.transpose(messages, (1, 0, 2))
    et = jnp.transpose(entropy, (1, 0)).reshape(_L, 1, _B)
    scalar_spec = pl.BlockSpec((1, 1), lambda l: (0, 0))
    out_t, sym_t, me_t, mn_t = pl.pallas_call(
        _body,
        grid=(_L,),
        in_specs=[
            scalar_spec,
            scalar_spec,
            scalar_spec,
            pl.BlockSpec((1, _B, _V), lambda l: (l, 0, 0)),
            pl.BlockSpec((1, 1, _B), lambda l: (l, 0, 0)),
        ],
        out_specs=[
            pl.BlockSpec((1, _V + 1, _B), lambda l: (l, 0, 0)),
            pl.BlockSpec((1, 1, _B), lambda l: (l, 0, 0)),
            pl.BlockSpec((1, 1, _B), lambda l: (0, 0, 0)),
            pl.BlockSpec((1, 1, _B), lambda l: (0, 0, 0)),
        ],
        out_shape=[
            jax.ShapeDtypeStruct((_L, _V + 1, _B), jnp.float32),
            jax.ShapeDtypeStruct((_L, 1, _B), jnp.float32),
            jax.ShapeDtypeStruct((1, 1, _B), jnp.float32),
            jax.ShapeDtypeStruct((1, 1, _B), jnp.float32),
        ],
    )(f, pe, c, mt, et)

    out = jnp.transpose(out_t, (2, 0, 1))
    sym = jnp.transpose(sym_t.reshape(_L, _B), (1, 0))
    return (out, me_t.reshape(_B), sym, mn_t.reshape(_B), entropy)
